# Initial kernel scaffold; baseline (speedup 1.0000x reference)
import jax
import jax.numpy as jnp
from jax.experimental import pallas as pl
from jax.experimental.pallas import tpu as pltpu

H = 1024
F = 3584
E = 8
T = 2048
KF = 512
NKF = F // KF


def _router_body(x_ref, gw_ref, rw_ref, idx_ref, comb_ref):
    x = x_ref[...]
    gw = gw_ref[...]
    logits = jax.lax.dot_general(
        x, gw, (((1,), (1,)), ((), ())),
        preferred_element_type=jnp.float32,
        precision=jax.lax.Precision.HIGHEST)
    m = jnp.max(logits, axis=1, keepdims=True)
    ex = jnp.exp(logits - m)
    probs = ex / jnp.sum(ex, axis=1, keepdims=True)
    lane = jax.lax.broadcasted_iota(jnp.int32, (T, E), 1)
    m1 = jnp.max(probs, axis=1, keepdims=True)
    a1 = jnp.min(jnp.where(probs == m1, lane, E), axis=1, keepdims=True)
    pm = jnp.where(lane == a1, -1.0, probs)
    m2 = jnp.max(pm, axis=1, keepdims=True)
    a2 = jnp.min(jnp.where(pm == m2, lane, E), axis=1, keepdims=True)
    s = m1 + m2
    w1 = m1 / s
    w2 = m2 / s
    rw_ref[...] = jnp.concatenate([w1, w2], axis=1)
    idx_ref[...] = jnp.concatenate([a1, a2], axis=1).astype(jnp.int32)
    comb_ref[...] = jnp.where(lane == a1, w1, 0.0) + jnp.where(lane == a2, w2, 0.0)


def _router(x, gate_w):
    return pl.pallas_call(
        _router_body,
        out_shape=[
            jax.ShapeDtypeStruct((T, 2), jnp.float32),
            jax.ShapeDtypeStruct((T, 2), jnp.int32),
            jax.ShapeDtypeStruct((T, E), jnp.float32),
        ],
    )(x, gate_w)


def _dense_body(x_ref, w1_ref, w3_ref, w2_ref, comb_ref, out_ref):
    e = pl.program_id(0)
    kf = pl.program_id(1)
    step = e * NKF + kf

    @pl.when(step == 0)
    def _():
        out_ref[...] = jnp.zeros_like(out_ref)

    x = x_ref[...]
    g = jax.lax.dot_general(x, w1_ref[0], (((1,), (1,)), ((), ())),
                            preferred_element_type=jnp.float32)
    u = jax.lax.dot_general(x, w3_ref[0], (((1,), (1,)), ((), ())),
                            preferred_element_type=jnp.float32)
    hh = g * (1.0 / (1.0 + jnp.exp(-g))) * u
    lane = jax.lax.broadcasted_iota(jnp.int32, (T, E), 1)
    c = jnp.sum(jnp.where(lane == e, comb_ref[...], 0.0), axis=1, keepdims=True)
    hh = hh * c
    out_ref[...] += jax.lax.dot_general(hh, w2_ref[0], (((1,), (1,)), ((), ())),
                                        preferred_element_type=jnp.float32)


def _dense(x, w1_w3, w2, comb):
    return pl.pallas_call(
        _dense_body,
        grid=(E, NKF),
        in_specs=[
            pl.BlockSpec((T, H), lambda e, kf: (0, 0)),
            pl.BlockSpec((1, KF, H), lambda e, kf: (e, kf, 0)),
            pl.BlockSpec((1, KF, H), lambda e, kf: (e, NKF + kf, 0)),
            pl.BlockSpec((1, H, KF), lambda e, kf: (e, 0, kf)),
            pl.BlockSpec((T, E), lambda e, kf: (0, 0)),
        ],
        out_specs=pl.BlockSpec((T, H), lambda e, kf: (0, 0)),
        out_shape=jax.ShapeDtypeStruct((T, H), jnp.float32),
        compiler_params=pltpu.CompilerParams(
            dimension_semantics=("arbitrary", "arbitrary")),
    )(x, w1_w3, w1_w3, w2, comb)


def kernel(hidden_states, gate_w, w1_w3, w2):
    rw, _eidx, comb = _router(hidden_states, gate_w)
    out = _dense(hidden_states, w1_w3, w2, comb)
    return out, rw


# speedup vs baseline: 1.4380x; 1.4380x over previous
import jax
import jax.numpy as jnp
from jax.experimental import pallas as pl
from jax.experimental.pallas import tpu as pltpu

H = 1024
F = 3584
E = 8
T = 2048
KF = 512
NKF = F // KF


def _router_body(x_ref, gw_ref, rw_ref, idx_ref, comb_ref):
    x = x_ref[...]
    gw = gw_ref[...]
    logits = jax.lax.dot_general(
        x, gw, (((1,), (1,)), ((), ())),
        preferred_element_type=jnp.float32)
    m = jnp.max(logits, axis=1, keepdims=True)
    ex = jnp.exp(logits - m)
    probs = ex / jnp.sum(ex, axis=1, keepdims=True)
    lane = jax.lax.broadcasted_iota(jnp.int32, (T, E), 1)
    m1 = jnp.max(probs, axis=1, keepdims=True)
    a1 = jnp.min(jnp.where(probs == m1, lane, E), axis=1, keepdims=True)
    pm = jnp.where(lane == a1, -1.0, probs)
    m2 = jnp.max(pm, axis=1, keepdims=True)
    a2 = jnp.min(jnp.where(pm == m2, lane, E), axis=1, keepdims=True)
    s = m1 + m2
    w1 = m1 / s
    w2 = m2 / s
    rw_ref[...] = jnp.concatenate([w1, w2], axis=1)
    idx_ref[...] = jnp.concatenate([a1, a2], axis=1).astype(jnp.int32)
    comb_ref[...] = jnp.where(lane == a1, w1, 0.0) + jnp.where(lane == a2, w2, 0.0)


def _router(x, gate_w):
    return pl.pallas_call(
        _router_body,
        out_shape=[
            jax.ShapeDtypeStruct((T, 2), jnp.float32),
            jax.ShapeDtypeStruct((T, 2), jnp.int32),
            jax.ShapeDtypeStruct((T, E), jnp.float32),
        ],
    )(x, gate_w)


def _dense_body(x_ref, w1_ref, w3_ref, w2_ref, comb_ref, out_ref):
    e = pl.program_id(0)
    kf = pl.program_id(1)
    step = e * NKF + kf

    @pl.when(step == 0)
    def _():
        out_ref[...] = jnp.zeros_like(out_ref)

    x = x_ref[...]
    g = jax.lax.dot_general(x, w1_ref[0], (((1,), (1,)), ((), ())),
                            preferred_element_type=jnp.float32)
    u = jax.lax.dot_general(x, w3_ref[0], (((1,), (1,)), ((), ())),
                            preferred_element_type=jnp.float32)
    hh = g * (1.0 / (1.0 + jnp.exp(-g))) * u
    lane = jax.lax.broadcasted_iota(jnp.int32, (T, E), 1)
    c = jnp.sum(jnp.where(lane == e, comb_ref[...], 0.0), axis=1, keepdims=True)
    hh = hh * c
    out_ref[...] += jax.lax.dot_general(hh, w2_ref[0], (((1,), (1,)), ((), ())),
                                        preferred_element_type=jnp.float32)


def _dense(x, w1_w3, w2, comb):
    return pl.pallas_call(
        _dense_body,
        grid=(E, NKF),
        in_specs=[
            pl.BlockSpec((T, H), lambda e, kf: (0, 0)),
            pl.BlockSpec((1, KF, H), lambda e, kf: (e, kf, 0)),
            pl.BlockSpec((1, KF, H), lambda e, kf: (e, NKF + kf, 0)),
            pl.BlockSpec((1, H, KF), lambda e, kf: (e, 0, kf)),
            pl.BlockSpec((T, E), lambda e, kf: (0, 0)),
        ],
        out_specs=pl.BlockSpec((T, H), lambda e, kf: (0, 0)),
        out_shape=jax.ShapeDtypeStruct((T, H), jnp.float32),
        compiler_params=pltpu.CompilerParams(
            dimension_semantics=("arbitrary", "arbitrary")),
    )(x, w1_w3, w1_w3, w2, comb)


def kernel(hidden_states, gate_w, w1_w3, w2):
    rw, _eidx, comb = _router(hidden_states, gate_w)
    out = _dense(hidden_states, w1_w3, w2, comb)
    return out, rw
